# baseline (speedup 1.0000x reference)
import jax
import jax.numpy as jnp
from jax.experimental import pallas as pl
from jax.experimental.pallas import tpu as pltpu

_ROWS = 16384
_COLS = 512
_TBLK = 4096
_TGRID = _ROWS // _TBLK


def _tc_body(p_ref, t_ref, o_ref):
    x = p_ref[...]
    t = t_ref[...]
    a = jnp.abs(x)
    sp = jnp.log(1.0 + jnp.exp(-a))
    loss = 0.5 * a - 0.125 * (x * t) + sp
    loss = jnp.where(t > 0.0, loss, 0.0)
    part = jnp.sum(loss)

    @pl.when(pl.program_id(0) == 0)
    def _init():
        o_ref[0] = 0.0

    o_ref[0] += part


def kernel(pred_map, target_map):
    p = pred_map.reshape(_ROWS, _COLS)
    t = target_map.reshape(_ROWS, _COLS)
    out = pl.pallas_call(
        _tc_body,
        grid=(_TGRID,),
        in_specs=[
            pl.BlockSpec((_TBLK, _COLS), lambda i: (i, 0)),
            pl.BlockSpec((_TBLK, _COLS), lambda i: (i, 0)),
        ],
        out_specs=pl.BlockSpec(memory_space=pltpu.SMEM),
        out_shape=jax.ShapeDtypeStruct((1,), jnp.float32),
    )(p, t)
    return out[0]


# speedup vs baseline: 1.1045x; 1.1045x over previous
import jax
import jax.numpy as jnp
from jax import lax
from jax.experimental import pallas as pl
from jax.experimental.pallas import tpu as pltpu

_ROWS = 16384
_COLS = 512
_CHR = 512
_NCH = _ROWS // _CHR
_NBUF = 4


def _tc_body(p_hbm, t_hbm, o_ref, pbuf, tbuf, psem, tsem):
    def p_copy(ci, slot):
        return pltpu.make_async_copy(
            p_hbm.at[pl.ds(ci * _CHR, _CHR), :], pbuf.at[slot], psem.at[slot])

    def t_copy(ci, slot):
        return pltpu.make_async_copy(
            t_hbm.at[pl.ds(ci * _CHR, _CHR), :], tbuf.at[slot], tsem.at[slot])

    for ci in range(_NBUF):
        p_copy(ci, ci).start()
        t_copy(ci, ci).start()

    o_ref[0] = 0.0

    def chunk(ci, carry):
        slot = lax.rem(ci, _NBUF)
        p_copy(ci, slot).wait()
        t_copy(ci, slot).wait()

        x = pbuf[slot]
        t = tbuf[slot]
        a = jnp.abs(x)
        sp = jnp.log(1.0 + jnp.exp(-a))
        loss = 0.5 * a - 0.125 * (x * t) + sp
        loss = jnp.where(t > 0.0, loss, 0.0)
        o_ref[0] += jnp.sum(loss)

        @pl.when(ci + _NBUF < _NCH)
        def _prefetch():
            p_copy(ci + _NBUF, slot).start()
            t_copy(ci + _NBUF, slot).start()

        return carry

    lax.fori_loop(0, _NCH, chunk, 0)


def kernel(pred_map, target_map):
    p = pred_map.reshape(_ROWS, _COLS)
    t = target_map.reshape(_ROWS, _COLS)
    out = pl.pallas_call(
        _tc_body,
        in_specs=[
            pl.BlockSpec(memory_space=pl.ANY),
            pl.BlockSpec(memory_space=pl.ANY),
        ],
        out_specs=pl.BlockSpec(memory_space=pltpu.SMEM),
        out_shape=jax.ShapeDtypeStruct((1,), jnp.float32),
        scratch_shapes=[
            pltpu.VMEM((_NBUF, _CHR, _COLS), jnp.float32),
            pltpu.VMEM((_NBUF, _CHR, _COLS), jnp.float32),
            pltpu.SemaphoreType.DMA((_NBUF,)),
            pltpu.SemaphoreType.DMA((_NBUF,)),
        ],
    )(p, t)
    return out[0]
